# baseline (speedup 1.0000x reference)
import jax
import jax.numpy as jnp
from jax.experimental import pallas as pl

N_NODES = 10000
N_EDGES = 160000
N_GRAPHS = 100
H = 32


def _gru_step(x, h, Wih, Whh, bih, bhh):
    gi = x @ Wih.T + bih
    gh = h @ Whh.T + bhh
    i_r, i_z, i_n = jnp.split(gi, 3, axis=-1)
    h_r, h_z, h_n = jnp.split(gh, 3, axis=-1)
    r = jax.nn.sigmoid(i_r + h_r)
    z = jax.nn.sigmoid(i_z + h_z)
    n = jnp.tanh(i_n + r * h_n)
    return (1.0 - z) * n + z * h


def _lstm_step(x, h, c, Wih, Whh, bih, bhh):
    g = x @ Wih.T + bih + h @ Whh.T + bhh
    i, f, gg, o = jnp.split(g, 4, axis=-1)
    c2 = jax.nn.sigmoid(f) * c + jax.nn.sigmoid(i) * jnp.tanh(gg)
    h2 = jax.nn.sigmoid(o) * jnp.tanh(c2)
    return h2, c2


def _segment_softmax(e, seg, num_segments):
    m = jax.ops.segment_max(e, seg, num_segments=num_segments)
    m = jnp.where(jnp.isfinite(m), m, 0.0)
    ex = jnp.exp(e - m[seg])
    den = jax.ops.segment_sum(ex, seg, num_segments=num_segments)
    return ex / (den[seg] + 1e-16)


def _lin0_pallas(x, W, b):
    def body(x_ref, w_ref, b_ref, o_ref):
        o_ref[...] = jax.nn.relu(x_ref[...] @ w_ref[...] + b_ref[...])

    return pl.pallas_call(
        body,
        out_shape=jax.ShapeDtypeStruct((x.shape[0], W.shape[1]), x.dtype),
        grid=(10,),
        in_specs=[
            pl.BlockSpec((x.shape[0] // 10, x.shape[1]), lambda i: (i, 0)),
            pl.BlockSpec((W.shape[0], W.shape[1]), lambda i: (0, 0)),
            pl.BlockSpec((1, W.shape[1]), lambda i: (0, 0)),
        ],
        out_specs=pl.BlockSpec((x.shape[0] // 10, W.shape[1]), lambda i: (i, 0)),
    )(x, W, b.reshape(1, -1))


def kernel(x, edge_index, edge_attr, batch, lin0_W, lin0_b, enn_W1, enn_b1, enn_W2, enn_b2, conv_root, conv_bias, gru_Wih, gru_Whh, gru_bih, gru_bhh, s2s_Wih, s2s_Whh, s2s_bih, s2s_bhh, mem_Wih, mem_Whh, mem_bih, mem_bhh, mlp_W1, mlp_b1, mlp_W2, mlp_b2):
    src = edge_index[0]
    dst = edge_index[1]
    out = _lin0_pallas(x, lin0_W, lin0_b)
    h = out
    ew = (jax.nn.relu(edge_attr @ enn_W1 + enn_b1) @ enn_W2 + enn_b2).reshape(-1, H, H)
    deg = jnp.maximum(jax.ops.segment_sum(jnp.ones((N_EDGES,), dtype=jnp.float32), dst, num_segments=N_NODES), 1.0)
    for _ in range(6):
        msg = jnp.einsum('ei,eio->eo', out[src], ew)
        agg = jax.ops.segment_sum(msg, dst, num_segments=N_NODES) / deg[:, None]
        m = jax.nn.relu(out @ conv_root + agg + conv_bias)
        h = _gru_step(m, h, gru_Wih, gru_Whh, gru_bih, gru_bhh)
        out = h
    qh = jnp.zeros((N_GRAPHS, H), dtype=jnp.float32)
    qc = jnp.zeros((N_GRAPHS, H), dtype=jnp.float32)
    q_star = jnp.zeros((N_GRAPHS, 2 * H), dtype=jnp.float32)
    for _ in range(6):
        qh, qc = _lstm_step(q_star, qh, qc, s2s_Wih, s2s_Whh, s2s_bih, s2s_bhh)
        e = jnp.sum(out * qh[batch], axis=-1)
        a = _segment_softmax(e, batch, N_GRAPHS)
        r = jax.ops.segment_sum(a[:, None] * out, batch, num_segments=N_GRAPHS)
        q_star = jnp.concatenate([qh, r], axis=-1)
    hx = jnp.zeros((N_GRAPHS, H), dtype=jnp.float32)
    cx = jnp.zeros((N_GRAPHS, H), dtype=jnp.float32)
    hx, cx = _lstm_step(q_star, hx, cx, mem_Wih, mem_Whh, mem_bih, mem_bhh)
    v = jax.nn.relu(hx @ mlp_W1 + mlp_b1) @ mlp_W2 + mlp_b2
    return v[None], hx[None], cx[None]


# speedup vs baseline: 1.9391x; 1.9391x over previous
import functools

import jax
import jax.numpy as jnp
from jax import lax
from jax.experimental import pallas as pl
from jax.experimental.pallas import tpu as pltpu
from jax.experimental.pallas import tpu_sc as plsc

N_NODES = 10000
N_EDGES = 160000
N_GRAPHS = 100
H = 32
NODE_DIM = 128
EDGE_DIM = 16

NW = 32
CHUNK = 128
NCH = 40
EPW = NCH * CHUNK
EP = NW * EPW
NPAD = 10240
NSL = NPAD // 16
EB = 512
GP = 128
LW = 128
SST = 64
NSP = NSL // SST


@functools.lru_cache(maxsize=1)
def _sc_gather_kernel():
    mesh = plsc.VectorSubcoreMesh(core_axis_name="c", subcore_axis_name="s")

    @functools.partial(
        pl.kernel,
        out_type=jax.ShapeDtypeStruct((EP, LW), jnp.float32),
        mesh=mesh,
        scratch_types=[
            pltpu.VMEM((NCH, CHUNK), jnp.int32),
            pltpu.VMEM((CHUNK, LW), jnp.float32),
            pltpu.SemaphoreType.DMA,
        ],
    )
    def body_fn(table_hbm, idx_hbm, u_hbm, idx_v, rows_v, sem):
        c = lax.axis_index("c")
        s = lax.axis_index("s")
        wid = c * 16 + s
        pltpu.sync_copy(idx_hbm.at[pl.ds(wid * NCH, NCH)], idx_v)

        def body(ch, carry):
            pltpu.async_copy(table_hbm.at[idx_v.at[ch]], rows_v, sem).wait()
            pltpu.sync_copy(rows_v, u_hbm.at[pl.ds(wid * EPW + ch * CHUNK, CHUNK)])
            return carry

        lax.fori_loop(0, NCH, body, 0)

    return body_fn


def _sc_gather(table, idx2d):
    return _sc_gather_kernel()(table, idx2d)


@functools.lru_cache(maxsize=1)
def _sc_scatter_kernel():
    mesh = plsc.VectorSubcoreMesh(core_axis_name="c", subcore_axis_name="s")

    @functools.partial(
        pl.kernel,
        out_type=jax.ShapeDtypeStruct((2, NPAD, LW), jnp.float32),
        mesh=mesh,
        scratch_types=[
            pltpu.VMEM_SHARED((NPAD, LW), jnp.float32),
            pltpu.VMEM((NCH, CHUNK), jnp.int32),
            pltpu.VMEM((SST, LW), jnp.float32),
            pltpu.VMEM((CHUNK, LW), jnp.float32),
        ],
    )
    def body_fn(msg_hbm, idx_hbm, zeros_hbm, agg_hbm, agg_sh, idx_v, stage_v, rows_v):
        c = lax.axis_index("c")
        s = lax.axis_index("s")
        wid = c * 16 + s

        def zero_body(p, carry):
            pltpu.sync_copy(zeros_hbm, stage_v)
            pltpu.sync_copy(stage_v, agg_sh.at[pl.ds((s * NSP + p) * SST, SST)])
            return carry

        lax.fori_loop(0, NSP, zero_body, 0)
        pltpu.sync_copy(idx_hbm.at[pl.ds(wid * NCH, NCH)], idx_v)
        plsc.subcore_barrier()

        def body(ch, carry):
            pltpu.sync_copy(msg_hbm.at[pl.ds(wid * EPW + ch * CHUNK, CHUNK)], rows_v)
            pltpu.sync_copy(rows_v, agg_sh.at[idx_v.at[ch]], add=True)
            return carry

        lax.fori_loop(0, NCH, body, 0)
        plsc.subcore_barrier()

        def out_body(p, carry):
            pltpu.sync_copy(agg_sh.at[pl.ds((s * NSP + p) * SST, SST)], stage_v)
            pltpu.sync_copy(stage_v, agg_hbm.at[c, pl.ds((s * NSP + p) * SST, SST)])
            return carry

        lax.fori_loop(0, NSP, out_body, 0)

    return body_fn


def _sc_scatter(msg, idx2d, zeros_sl):
    return _sc_scatter_kernel()(msg, idx2d, zeros_sl)


def _eye(n):
    r = lax.broadcasted_iota(jnp.int32, (n, n), 0)
    c = lax.broadcasted_iota(jnp.int32, (n, n), 1)
    return (r == c).astype(jnp.float32)


def _rep4(o):
    return jnp.concatenate([o, o, o, o], axis=1)


def _tc_prep(x_p, W, b_row):
    def body(x_ref, w_ref, b_ref, o_ref):
        o = jax.nn.relu(
            jnp.dot(x_ref[...], w_ref[...], preferred_element_type=jnp.float32)
            + b_ref[...]
        )
        o_ref[...] = _rep4(o)

    return pl.pallas_call(
        body,
        out_shape=jax.ShapeDtypeStruct((NPAD, LW), jnp.float32),
    )(x_p, W, b_row)


def _tc_edget(ea_p, W1, b1_row):
    blk = 2048

    def body(ea_ref, w_ref, b_ref, o_ref):
        t = jax.nn.relu(
            jnp.dot(ea_ref[...], w_ref[...], preferred_element_type=jnp.float32)
            + b_ref[...]
        )
        tT = lax.dot_general(_eye(H), t, (((1,), (1,)), ((), ())),
                             preferred_element_type=jnp.float32)
        ones = jnp.ones((1, blk), jnp.float32)
        zeros = jnp.zeros((7, blk), jnp.float32)
        o_ref[...] = jnp.concatenate([tT, ones, zeros], axis=0)

    return pl.pallas_call(
        body,
        out_shape=jax.ShapeDtypeStruct((40, EP), jnp.float32),
        grid=(EP // blk,),
        in_specs=[
            pl.BlockSpec((blk, EDGE_DIM), lambda j: (j, 0)),
            pl.BlockSpec((EDGE_DIM, H), lambda j: (0, 0)),
            pl.BlockSpec((1, H), lambda j: (0, 0)),
        ],
        out_specs=pl.BlockSpec((40, blk), lambda j: (0, j)),
    )(ea_p, W1, b1_row)


def _tc_msg(tT, u, W2b):
    def body(t_ref, u_ref, w_ref, o_ref):
        ew = jnp.dot(w_ref[...], t_ref[...], preferred_element_type=jnp.float32)
        ub = u_ref[:, 0:H]
        uT = lax.dot_general(_eye(H), ub, (((1,), (1,)), ((), ())),
                             preferred_element_type=jnp.float32)
        acc = uT[0:1, :] * ew[0:H, :]
        for i in range(1, H):
            acc = acc + uT[i:i + 1, :] * ew[i * H:(i + 1) * H, :]
        msg = lax.dot_general(acc, _eye(H), (((0,), (0,)), ((), ())),
                              preferred_element_type=jnp.float32)
        o_ref[...] = jnp.concatenate(
            [msg, jnp.zeros((EB, LW - H), jnp.float32)], axis=1)

    return pl.pallas_call(
        body,
        out_shape=jax.ShapeDtypeStruct((EP, LW), jnp.float32),
        grid=(EP // EB,),
        in_specs=[
            pl.BlockSpec((40, EB), lambda j: (0, j)),
            pl.BlockSpec((EB, LW), lambda j: (j, 0)),
            pl.BlockSpec((H * H, 40), lambda j: (0, 0)),
        ],
        out_specs=pl.BlockSpec((EB, LW), lambda j: (j, 0)),
    )(tT, u, W2b)


def _tc_update(out4, h4, aggP, degP, conv_root, cb_row, WihT, WhhT, bih_row, bhh_row):
    def body(o_ref, h_ref, a_ref, d_ref, cr_ref, cb_ref, wi_ref, wh_ref, bi_ref, bh_ref, out_ref):
        agg = a_ref[0, :, 0:H] + a_ref[1, :, 0:H]
        deg = d_ref[0, :, 0:1] + d_ref[1, :, 0:1]
        rdeg = 1.0 / jnp.maximum(deg, 1.0)
        o = o_ref[:, 0:H]
        hh = h_ref[:, 0:H]
        m = jax.nn.relu(
            jnp.dot(o, cr_ref[...], preferred_element_type=jnp.float32)
            + agg * rdeg + cb_ref[...]
        )
        gi = jnp.dot(m, wi_ref[...], preferred_element_type=jnp.float32) + bi_ref[...]
        gh = jnp.dot(hh, wh_ref[...], preferred_element_type=jnp.float32) + bh_ref[...]
        r = jax.nn.sigmoid(gi[:, 0:H] + gh[:, 0:H])
        z = jax.nn.sigmoid(gi[:, H:2 * H] + gh[:, H:2 * H])
        n = jnp.tanh(gi[:, 2 * H:3 * H] + r * gh[:, 2 * H:3 * H])
        out_ref[...] = _rep4((1.0 - z) * n + z * hh)

    return pl.pallas_call(
        body,
        out_shape=jax.ShapeDtypeStruct((NPAD, LW), jnp.float32),
    )(out4, h4, aggP, degP, conv_root, cb_row, WihT, WhhT, bih_row, bhh_row)


def _tc_final(out4, batch_col, batch_row,
              s2s_WihT, s2s_WhhT, s2s_bih_row, s2s_bhh_row,
              mem_WihT, mem_WhhT, mem_bih_row, mem_bhh_row,
              mlp_W1, mlp_b1_row, mlp_W2, mlp_b2_row):
    def body(o_ref, bc_ref, br_ref,
             swi_ref, swh_ref, sbi_ref, sbh_ref,
             mwi_ref, mwh_ref, mbi_ref, mbh_ref,
             w1_ref, b1_ref, w2_ref, b2_ref,
             v_ref, hx_ref, cx_ref):
        o = o_ref[:, 0:H]
        bc = bc_ref[...]
        gcol = lax.broadcasted_iota(jnp.int32, (NPAD, GP), 1)
        OH = (bc == gcol).astype(jnp.float32)
        br = br_ref[...]
        grow = lax.broadcasted_iota(jnp.int32, (GP, NPAD), 0)
        OHT = (br == grow).astype(jnp.float32)

        qh = jnp.zeros((GP, H), jnp.float32)
        qc = jnp.zeros((GP, H), jnp.float32)
        q_star = jnp.zeros((GP, 2 * H), jnp.float32)
        for _ in range(6):
            g = (jnp.dot(q_star, swi_ref[...], preferred_element_type=jnp.float32)
                 + sbi_ref[...]
                 + jnp.dot(qh, swh_ref[...], preferred_element_type=jnp.float32)
                 + sbh_ref[...])
            ig = jax.nn.sigmoid(g[:, 0:H])
            fg = jax.nn.sigmoid(g[:, H:2 * H])
            gg = jnp.tanh(g[:, 2 * H:3 * H])
            og = jax.nn.sigmoid(g[:, 3 * H:4 * H])
            qc = fg * qc + ig * gg
            qh = og * jnp.tanh(qc)

            qhb = jnp.dot(OH, qh, preferred_element_type=jnp.float32)
            e_col = jnp.sum(o * qhb, axis=1, keepdims=True)
            masked = jnp.where(OH > 0.0, e_col, -1e30)
            m_row = jnp.max(masked, axis=0, keepdims=True)
            m_row = jnp.where(m_row < -1e29, 0.0, m_row)
            mb = jnp.sum(OH * m_row, axis=1, keepdims=True)
            ex = jnp.exp(e_col - mb)
            den_row = jnp.sum(OH * ex, axis=0, keepdims=True)
            denb = jnp.sum(OH * den_row, axis=1, keepdims=True)
            a_col = ex / (denb + 1e-16)
            r = jnp.dot(OHT, a_col * o, preferred_element_type=jnp.float32)
            q_star = jnp.concatenate([qh, r], axis=1)

        g = (jnp.dot(q_star, mwi_ref[...], preferred_element_type=jnp.float32)
             + mbi_ref[...] + mbh_ref[...])
        ig = jax.nn.sigmoid(g[:, 0:H])
        fg = jax.nn.sigmoid(g[:, H:2 * H])
        gg = jnp.tanh(g[:, 2 * H:3 * H])
        og = jax.nn.sigmoid(g[:, 3 * H:4 * H])
        cx = ig * gg
        hx = og * jnp.tanh(cx)
        v = (jnp.dot(jax.nn.relu(
                jnp.dot(hx, w1_ref[...], preferred_element_type=jnp.float32)
                + b1_ref[...]),
             w2_ref[...], preferred_element_type=jnp.float32) + b2_ref[...])
        v_ref[...] = v[0:N_GRAPHS, :]
        hx_ref[...] = hx[0:N_GRAPHS, :]
        cx_ref[...] = cx[0:N_GRAPHS, :]

    return pl.pallas_call(
        body,
        out_shape=(
            jax.ShapeDtypeStruct((N_GRAPHS, 1), jnp.float32),
            jax.ShapeDtypeStruct((N_GRAPHS, H), jnp.float32),
            jax.ShapeDtypeStruct((N_GRAPHS, H), jnp.float32),
        ),
    )(out4, batch_col, batch_row,
      s2s_WihT, s2s_WhhT, s2s_bih_row, s2s_bhh_row,
      mem_WihT, mem_WhhT, mem_bih_row, mem_bhh_row,
      mlp_W1, mlp_b1_row, mlp_W2, mlp_b2_row)


def kernel(x, edge_index, edge_attr, batch, lin0_W, lin0_b, enn_W1, enn_b1, enn_W2, enn_b2, conv_root, conv_bias, gru_Wih, gru_Whh, gru_bih, gru_bhh, s2s_Wih, s2s_Whh, s2s_bih, s2s_bhh, mem_Wih, mem_Whh, mem_bih, mem_bhh, mlp_W1, mlp_b1, mlp_W2, mlp_b2):
    f32 = jnp.float32
    src = edge_index[0]
    dst = edge_index[1]
    pad_e = EP - N_EDGES
    src_p = jnp.concatenate([src, jnp.zeros((pad_e,), jnp.int32)]).reshape(NW * NCH, CHUNK)
    trash = N_NODES + (jnp.arange(pad_e, dtype=jnp.int32) % (NPAD - N_NODES))
    dst_p = jnp.concatenate([dst, trash]).reshape(NW * NCH, CHUNK)
    x_p = jnp.concatenate([x, jnp.zeros((NPAD - N_NODES, NODE_DIM), f32)])
    ea_p = jnp.concatenate([edge_attr, jnp.zeros((pad_e, EDGE_DIM), f32)])
    pad_n = NPAD - N_NODES
    batch_p = jnp.concatenate(
        [batch, N_GRAPHS + (jnp.arange(pad_n, dtype=jnp.int32) % (GP - N_GRAPHS))])
    batch_col = batch_p[:, None]
    batch_row = batch_p[None, :]

    W2b = jnp.concatenate(
        [enn_W2.T, enn_b2[:, None], jnp.zeros((H * H, 7), f32)], axis=1)
    ones_ep = jnp.ones((EP, LW), f32)
    zeros_sl = jnp.zeros((SST, LW), f32)

    out = _tc_prep(x_p, lin0_W, lin0_b.reshape(1, H))
    tT = _tc_edget(ea_p, enn_W1, enn_b1.reshape(1, H))
    degP = _sc_scatter(ones_ep, dst_p, zeros_sl)

    h = out
    gWihT = gru_Wih.T
    gWhhT = gru_Whh.T
    for _ in range(6):
        u = _sc_gather(out, src_p)
        msg = _tc_msg(tT, u, W2b)
        aggP = _sc_scatter(msg, dst_p, zeros_sl)
        out = _tc_update(out, h, aggP, degP, conv_root, conv_bias.reshape(1, H),
                         gWihT, gWhhT, gru_bih.reshape(1, 3 * H), gru_bhh.reshape(1, 3 * H))
        h = out

    v, hx, cx = _tc_final(
        out, batch_col, batch_row,
        s2s_Wih.T, s2s_Whh.T, s2s_bih.reshape(1, 4 * H), s2s_bhh.reshape(1, 4 * H),
        mem_Wih.T, mem_Whh.T, mem_bih.reshape(1, 4 * H), mem_bhh.reshape(1, 4 * H),
        mlp_W1, mlp_b1.reshape(1, H), mlp_W2, mlp_b2.reshape(1, 1))
    return v[None], hx[None], cx[None]


# speedup vs baseline: 2.7381x; 1.4120x over previous
import functools

import jax
import jax.numpy as jnp
from jax import lax
from jax.experimental import pallas as pl
from jax.experimental.pallas import tpu as pltpu
from jax.experimental.pallas import tpu_sc as plsc

N_NODES = 10000
N_EDGES = 160000
N_GRAPHS = 100
H = 32
NODE_DIM = 128
EDGE_DIM = 16

NW = 32
CHUNK = 128
NCH = 40
EPW = NCH * CHUNK
EP = NW * EPW
NPAD = 10240
NSL = NPAD // 16
EB = 512
GP = 128
LW = 128
SST = 64
NSP = NSL // SST


@functools.lru_cache(maxsize=1)
def _sc_gather_kernel():
    mesh = plsc.VectorSubcoreMesh(core_axis_name="c", subcore_axis_name="s")

    @functools.partial(
        pl.kernel,
        out_type=jax.ShapeDtypeStruct((EP, LW), jnp.float32),
        mesh=mesh,
        scratch_types=[
            pltpu.VMEM_SHARED((NPAD, LW), jnp.float32),
            pltpu.VMEM((NCH, CHUNK), jnp.int32),
            pltpu.VMEM((SST, LW), jnp.float32),
            pltpu.VMEM((CHUNK, LW), jnp.float32),
            pltpu.SemaphoreType.DMA,
        ],
    )
    def body_fn(table_hbm, idx_hbm, u_hbm, tbl_sh, idx_v, stage_v, rows_v, sem):
        c = lax.axis_index("c")
        s = lax.axis_index("s")
        wid = c * 16 + s

        def stage_body(p, carry):
            pltpu.sync_copy(table_hbm.at[pl.ds((s * NSP + p) * SST, SST)], stage_v)
            pltpu.sync_copy(stage_v, tbl_sh.at[pl.ds((s * NSP + p) * SST, SST)])
            return carry

        lax.fori_loop(0, NSP, stage_body, 0)
        pltpu.sync_copy(idx_hbm.at[pl.ds(wid * NCH, NCH)], idx_v)
        plsc.subcore_barrier()

        def body(ch, carry):
            pltpu.async_copy(tbl_sh.at[idx_v.at[ch]], rows_v, sem).wait()
            pltpu.sync_copy(rows_v, u_hbm.at[pl.ds(wid * EPW + ch * CHUNK, CHUNK)])
            return carry

        lax.fori_loop(0, NCH, body, 0)

    return body_fn


def _sc_gather(table, idx2d):
    return _sc_gather_kernel()(table, idx2d)


@functools.lru_cache(maxsize=1)
def _sc_scatter_kernel():
    mesh = plsc.VectorSubcoreMesh(core_axis_name="c", subcore_axis_name="s")

    @functools.partial(
        pl.kernel,
        out_type=jax.ShapeDtypeStruct((2, NPAD, LW), jnp.float32),
        mesh=mesh,
        scratch_types=[
            pltpu.VMEM_SHARED((NPAD, LW), jnp.float32),
            pltpu.VMEM((NCH, CHUNK), jnp.int32),
            pltpu.VMEM((SST, LW), jnp.float32),
            pltpu.VMEM((CHUNK, LW), jnp.float32),
        ],
    )
    def body_fn(msg_hbm, idx_hbm, zeros_hbm, agg_hbm, agg_sh, idx_v, stage_v, rows_v):
        c = lax.axis_index("c")
        s = lax.axis_index("s")
        wid = c * 16 + s

        def zero_body(p, carry):
            pltpu.sync_copy(zeros_hbm, stage_v)
            pltpu.sync_copy(stage_v, agg_sh.at[pl.ds((s * NSP + p) * SST, SST)])
            return carry

        lax.fori_loop(0, NSP, zero_body, 0)
        pltpu.sync_copy(idx_hbm.at[pl.ds(wid * NCH, NCH)], idx_v)
        plsc.subcore_barrier()

        def body(ch, carry):
            pltpu.sync_copy(msg_hbm.at[pl.ds(wid * EPW + ch * CHUNK, CHUNK)], rows_v)
            pltpu.sync_copy(rows_v, agg_sh.at[idx_v.at[ch]], add=True)
            return carry

        lax.fori_loop(0, NCH, body, 0)
        plsc.subcore_barrier()

        def out_body(p, carry):
            pltpu.sync_copy(agg_sh.at[pl.ds((s * NSP + p) * SST, SST)], stage_v)
            pltpu.sync_copy(stage_v, agg_hbm.at[c, pl.ds((s * NSP + p) * SST, SST)])
            return carry

        lax.fori_loop(0, NSP, out_body, 0)

    return body_fn


def _sc_scatter(msg, idx2d, zeros_sl):
    return _sc_scatter_kernel()(msg, idx2d, zeros_sl)


def _eye(n):
    r = lax.broadcasted_iota(jnp.int32, (n, n), 0)
    c = lax.broadcasted_iota(jnp.int32, (n, n), 1)
    return (r == c).astype(jnp.float32)


def _rep4(o):
    return jnp.concatenate([o, o, o, o], axis=1)


def _tc_prep(x_p, W, b_row):
    def body(x_ref, w_ref, b_ref, o_ref):
        o = jax.nn.relu(
            jnp.dot(x_ref[...], w_ref[...], preferred_element_type=jnp.float32)
            + b_ref[...]
        )
        o_ref[...] = _rep4(o)

    return pl.pallas_call(
        body,
        out_shape=jax.ShapeDtypeStruct((NPAD, LW), jnp.float32),
    )(x_p, W, b_row)


def _tc_edget(ea_p, W1, b1_row):
    blk = 2048

    def body(ea_ref, w_ref, b_ref, o_ref):
        t = jax.nn.relu(
            jnp.dot(ea_ref[...], w_ref[...], preferred_element_type=jnp.float32)
            + b_ref[...]
        )
        tT = lax.dot_general(_eye(H), t, (((1,), (1,)), ((), ())),
                             preferred_element_type=jnp.float32)
        ones = jnp.ones((1, blk), jnp.float32)
        zeros = jnp.zeros((7, blk), jnp.float32)
        o_ref[...] = jnp.concatenate([tT, ones, zeros], axis=0)

    return pl.pallas_call(
        body,
        out_shape=jax.ShapeDtypeStruct((40, EP), jnp.float32),
        grid=(EP // blk,),
        in_specs=[
            pl.BlockSpec((blk, EDGE_DIM), lambda j: (j, 0)),
            pl.BlockSpec((EDGE_DIM, H), lambda j: (0, 0)),
            pl.BlockSpec((1, H), lambda j: (0, 0)),
        ],
        out_specs=pl.BlockSpec((40, blk), lambda j: (0, j)),
    )(ea_p, W1, b1_row)


def _tc_msg(tT, u, W2b):
    def body(t_ref, u_ref, w_ref, o_ref):
        ew = jnp.dot(w_ref[...], t_ref[...], preferred_element_type=jnp.float32)
        ub = u_ref[:, 0:H]
        uT = lax.dot_general(_eye(H), ub, (((1,), (1,)), ((), ())),
                             preferred_element_type=jnp.float32)
        acc = uT[0:1, :] * ew[0:H, :]
        for i in range(1, H):
            acc = acc + uT[i:i + 1, :] * ew[i * H:(i + 1) * H, :]
        msg = lax.dot_general(acc, _eye(H), (((0,), (0,)), ((), ())),
                              preferred_element_type=jnp.float32)
        o_ref[...] = jnp.concatenate(
            [msg, jnp.zeros((EB, LW - H), jnp.float32)], axis=1)

    return pl.pallas_call(
        body,
        out_shape=jax.ShapeDtypeStruct((EP, LW), jnp.float32),
        grid=(EP // EB,),
        in_specs=[
            pl.BlockSpec((40, EB), lambda j: (0, j)),
            pl.BlockSpec((EB, LW), lambda j: (j, 0)),
            pl.BlockSpec((H * H, 40), lambda j: (0, 0)),
        ],
        out_specs=pl.BlockSpec((EB, LW), lambda j: (j, 0)),
    )(tT, u, W2b)


def _tc_update(out4, h4, aggP, degP, conv_root, cb_row, WihT, WhhT, bih_row, bhh_row):
    def body(o_ref, h_ref, a_ref, d_ref, cr_ref, cb_ref, wi_ref, wh_ref, bi_ref, bh_ref, out_ref):
        agg = a_ref[0, :, 0:H] + a_ref[1, :, 0:H]
        deg = d_ref[0, :, 0:1] + d_ref[1, :, 0:1]
        rdeg = 1.0 / jnp.maximum(deg, 1.0)
        o = o_ref[:, 0:H]
        hh = h_ref[:, 0:H]
        m = jax.nn.relu(
            jnp.dot(o, cr_ref[...], preferred_element_type=jnp.float32)
            + agg * rdeg + cb_ref[...]
        )
        gi = jnp.dot(m, wi_ref[...], preferred_element_type=jnp.float32) + bi_ref[...]
        gh = jnp.dot(hh, wh_ref[...], preferred_element_type=jnp.float32) + bh_ref[...]
        r = jax.nn.sigmoid(gi[:, 0:H] + gh[:, 0:H])
        z = jax.nn.sigmoid(gi[:, H:2 * H] + gh[:, H:2 * H])
        n = jnp.tanh(gi[:, 2 * H:3 * H] + r * gh[:, 2 * H:3 * H])
        out_ref[...] = _rep4((1.0 - z) * n + z * hh)

    return pl.pallas_call(
        body,
        out_shape=jax.ShapeDtypeStruct((NPAD, LW), jnp.float32),
    )(out4, h4, aggP, degP, conv_root, cb_row, WihT, WhhT, bih_row, bhh_row)


def _tc_final(out4, batch_col, batch_row,
              s2s_WihT, s2s_WhhT, s2s_bih_row, s2s_bhh_row,
              mem_WihT, mem_WhhT, mem_bih_row, mem_bhh_row,
              mlp_W1, mlp_b1_row, mlp_W2, mlp_b2_row):
    def body(o_ref, bc_ref, br_ref,
             swi_ref, swh_ref, sbi_ref, sbh_ref,
             mwi_ref, mwh_ref, mbi_ref, mbh_ref,
             w1_ref, b1_ref, w2_ref, b2_ref,
             v_ref, hx_ref, cx_ref):
        o = o_ref[:, 0:H]
        bc = bc_ref[...]
        gcol = lax.broadcasted_iota(jnp.int32, (NPAD, GP), 1)
        OH = (bc == gcol).astype(jnp.float32)
        br = br_ref[...]
        grow = lax.broadcasted_iota(jnp.int32, (GP, NPAD), 0)
        OHT = (br == grow).astype(jnp.float32)

        qh = jnp.zeros((GP, H), jnp.float32)
        qc = jnp.zeros((GP, H), jnp.float32)
        q_star = jnp.zeros((GP, 2 * H), jnp.float32)
        for _ in range(6):
            g = (jnp.dot(q_star, swi_ref[...], preferred_element_type=jnp.float32)
                 + sbi_ref[...]
                 + jnp.dot(qh, swh_ref[...], preferred_element_type=jnp.float32)
                 + sbh_ref[...])
            ig = jax.nn.sigmoid(g[:, 0:H])
            fg = jax.nn.sigmoid(g[:, H:2 * H])
            gg = jnp.tanh(g[:, 2 * H:3 * H])
            og = jax.nn.sigmoid(g[:, 3 * H:4 * H])
            qc = fg * qc + ig * gg
            qh = og * jnp.tanh(qc)

            qhb = jnp.dot(OH, qh, preferred_element_type=jnp.float32)
            e_col = jnp.sum(o * qhb, axis=1, keepdims=True)
            masked = jnp.where(OH > 0.0, e_col, -1e30)
            m_row = jnp.max(masked, axis=0, keepdims=True)
            m_row = jnp.where(m_row < -1e29, 0.0, m_row)
            mb = jnp.sum(OH * m_row, axis=1, keepdims=True)
            ex = jnp.exp(e_col - mb)
            den_row = jnp.sum(OH * ex, axis=0, keepdims=True)
            denb = jnp.sum(OH * den_row, axis=1, keepdims=True)
            a_col = ex / (denb + 1e-16)
            r = jnp.dot(OHT, a_col * o, preferred_element_type=jnp.float32)
            q_star = jnp.concatenate([qh, r], axis=1)

        g = (jnp.dot(q_star, mwi_ref[...], preferred_element_type=jnp.float32)
             + mbi_ref[...] + mbh_ref[...])
        ig = jax.nn.sigmoid(g[:, 0:H])
        fg = jax.nn.sigmoid(g[:, H:2 * H])
        gg = jnp.tanh(g[:, 2 * H:3 * H])
        og = jax.nn.sigmoid(g[:, 3 * H:4 * H])
        cx = ig * gg
        hx = og * jnp.tanh(cx)
        v = (jnp.dot(jax.nn.relu(
                jnp.dot(hx, w1_ref[...], preferred_element_type=jnp.float32)
                + b1_ref[...]),
             w2_ref[...], preferred_element_type=jnp.float32) + b2_ref[...])
        v_ref[...] = v[0:N_GRAPHS, :]
        hx_ref[...] = hx[0:N_GRAPHS, :]
        cx_ref[...] = cx[0:N_GRAPHS, :]

    return pl.pallas_call(
        body,
        out_shape=(
            jax.ShapeDtypeStruct((N_GRAPHS, 1), jnp.float32),
            jax.ShapeDtypeStruct((N_GRAPHS, H), jnp.float32),
            jax.ShapeDtypeStruct((N_GRAPHS, H), jnp.float32),
        ),
    )(out4, batch_col, batch_row,
      s2s_WihT, s2s_WhhT, s2s_bih_row, s2s_bhh_row,
      mem_WihT, mem_WhhT, mem_bih_row, mem_bhh_row,
      mlp_W1, mlp_b1_row, mlp_W2, mlp_b2_row)


def kernel(x, edge_index, edge_attr, batch, lin0_W, lin0_b, enn_W1, enn_b1, enn_W2, enn_b2, conv_root, conv_bias, gru_Wih, gru_Whh, gru_bih, gru_bhh, s2s_Wih, s2s_Whh, s2s_bih, s2s_bhh, mem_Wih, mem_Whh, mem_bih, mem_bhh, mlp_W1, mlp_b1, mlp_W2, mlp_b2):
    f32 = jnp.float32
    src = edge_index[0]
    dst = edge_index[1]
    pad_e = EP - N_EDGES
    src_p = jnp.concatenate([src, jnp.zeros((pad_e,), jnp.int32)]).reshape(NW * NCH, CHUNK)
    trash = N_NODES + (jnp.arange(pad_e, dtype=jnp.int32) % (NPAD - N_NODES))
    dst_p = jnp.concatenate([dst, trash]).reshape(NW * NCH, CHUNK)
    x_p = jnp.concatenate([x, jnp.zeros((NPAD - N_NODES, NODE_DIM), f32)])
    ea_p = jnp.concatenate([edge_attr, jnp.zeros((pad_e, EDGE_DIM), f32)])
    pad_n = NPAD - N_NODES
    batch_p = jnp.concatenate(
        [batch, N_GRAPHS + (jnp.arange(pad_n, dtype=jnp.int32) % (GP - N_GRAPHS))])
    batch_col = batch_p[:, None]
    batch_row = batch_p[None, :]

    W2b = jnp.concatenate(
        [enn_W2.T, enn_b2[:, None], jnp.zeros((H * H, 7), f32)], axis=1)
    ones_ep = jnp.ones((EP, LW), f32)
    zeros_sl = jnp.zeros((SST, LW), f32)

    out = _tc_prep(x_p, lin0_W, lin0_b.reshape(1, H))
    tT = _tc_edget(ea_p, enn_W1, enn_b1.reshape(1, H))
    degP = _sc_scatter(ones_ep, dst_p, zeros_sl)

    h = out
    gWihT = gru_Wih.T
    gWhhT = gru_Whh.T
    for _ in range(6):
        u = _sc_gather(out, src_p)
        msg = _tc_msg(tT, u, W2b)
        aggP = _sc_scatter(msg, dst_p, zeros_sl)
        out = _tc_update(out, h, aggP, degP, conv_root, conv_bias.reshape(1, H),
                         gWihT, gWhhT, gru_bih.reshape(1, 3 * H), gru_bhh.reshape(1, 3 * H))
        h = out

    v, hx, cx = _tc_final(
        out, batch_col, batch_row,
        s2s_Wih.T, s2s_Whh.T, s2s_bih.reshape(1, 4 * H), s2s_bhh.reshape(1, 4 * H),
        mem_Wih.T, mem_Whh.T, mem_bih.reshape(1, 4 * H), mem_bhh.reshape(1, 4 * H),
        mlp_W1, mlp_b1.reshape(1, H), mlp_W2, mlp_b2.reshape(1, 1))
    return v[None], hx[None], cx[None]


# speedup vs baseline: 3.0008x; 1.0959x over previous
import functools

import jax
import jax.numpy as jnp
from jax import lax
from jax.experimental import pallas as pl
from jax.experimental.pallas import tpu as pltpu
from jax.experimental.pallas import tpu_sc as plsc

N_NODES = 10000
N_EDGES = 160000
N_GRAPHS = 100
H = 32
NODE_DIM = 128
EDGE_DIM = 16

NW = 32
CHUNK = 128
NCH = 40
EPW = NCH * CHUNK
EP = NW * EPW
NPAD = 10240
NSL = NPAD // 16
EB = 512
GP = 128
LW = 128
SST = 64
NSP = NSL // SST


@functools.lru_cache(maxsize=1)
def _sc_gather_kernel():
    mesh = plsc.VectorSubcoreMesh(core_axis_name="c", subcore_axis_name="s")

    @functools.partial(
        pl.kernel,
        out_type=jax.ShapeDtypeStruct((EP, LW), jnp.float32),
        mesh=mesh,
        scratch_types=[
            pltpu.VMEM_SHARED((NPAD, LW), jnp.float32),
            pltpu.VMEM((NCH, CHUNK), jnp.int32),
            pltpu.VMEM((SST, LW), jnp.float32),
            pltpu.VMEM((CHUNK, LW), jnp.float32),
            pltpu.VMEM((CHUNK, LW), jnp.float32),
            pltpu.SemaphoreType.DMA,
            pltpu.SemaphoreType.DMA,
            pltpu.SemaphoreType.DMA,
            pltpu.SemaphoreType.DMA,
        ],
    )
    def body_fn(table_hbm, idx_hbm, u_hbm, tbl_sh, idx_v, stage_v,
                rows_a, rows_b, gsem_a, gsem_b, wsem_a, wsem_b):
        c = lax.axis_index("c")
        s = lax.axis_index("s")
        wid = c * 16 + s

        def stage_body(p, carry):
            pltpu.sync_copy(table_hbm.at[pl.ds((s * NSP + p) * SST, SST)], stage_v)
            pltpu.sync_copy(stage_v, tbl_sh.at[pl.ds((s * NSP + p) * SST, SST)])
            return carry

        lax.fori_loop(0, NSP, stage_body, 0)
        pltpu.sync_copy(idx_hbm.at[pl.ds(wid * NCH, NCH)], idx_v)
        plsc.subcore_barrier()

        bufs = (rows_a, rows_b)
        gsems = (gsem_a, gsem_b)
        wsems = (wsem_a, wsem_b)
        g = [None] * NCH
        w = [None] * NCH
        g[0] = pltpu.async_copy(tbl_sh.at[idx_v.at[0]], bufs[0], gsems[0])
        for ch in range(NCH):
            b = ch % 2
            nb = (ch + 1) % 2
            if ch + 1 < NCH:
                if ch >= 1:
                    w[ch - 1].wait()
                g[ch + 1] = pltpu.async_copy(tbl_sh.at[idx_v.at[ch + 1]], bufs[nb], gsems[nb])
            g[ch].wait()
            w[ch] = pltpu.async_copy(
                bufs[b], u_hbm.at[pl.ds(wid * EPW + ch * CHUNK, CHUNK)], wsems[b])
        w[NCH - 2].wait()
        w[NCH - 1].wait()

    return body_fn


def _sc_gather(table, idx2d):
    return _sc_gather_kernel()(table, idx2d)


@functools.lru_cache(maxsize=1)
def _sc_scatter_kernel():
    mesh = plsc.VectorSubcoreMesh(core_axis_name="c", subcore_axis_name="s")

    @functools.partial(
        pl.kernel,
        out_type=jax.ShapeDtypeStruct((2, NPAD, LW), jnp.float32),
        mesh=mesh,
        scratch_types=[
            pltpu.VMEM_SHARED((NPAD, LW), jnp.float32),
            pltpu.VMEM((NCH, CHUNK), jnp.int32),
            pltpu.VMEM((SST, LW), jnp.float32),
            pltpu.VMEM((CHUNK, LW), jnp.float32),
            pltpu.VMEM((CHUNK, LW), jnp.float32),
            pltpu.SemaphoreType.DMA,
            pltpu.SemaphoreType.DMA,
            pltpu.SemaphoreType.DMA,
            pltpu.SemaphoreType.DMA,
        ],
    )
    def body_fn(msg_hbm, idx_hbm, zeros_hbm, agg_hbm, agg_sh, idx_v, stage_v,
                rows_a, rows_b, rsem_a, rsem_b, ssem_a, ssem_b):
        c = lax.axis_index("c")
        s = lax.axis_index("s")
        wid = c * 16 + s

        def zero_body(p, carry):
            pltpu.sync_copy(zeros_hbm, stage_v)
            pltpu.sync_copy(stage_v, agg_sh.at[pl.ds((s * NSP + p) * SST, SST)])
            return carry

        lax.fori_loop(0, NSP, zero_body, 0)
        pltpu.sync_copy(idx_hbm.at[pl.ds(wid * NCH, NCH)], idx_v)
        plsc.subcore_barrier()

        bufs = (rows_a, rows_b)
        rsems = (rsem_a, rsem_b)
        ssems = (ssem_a, ssem_b)
        r = [None] * NCH
        sc = [None] * NCH
        r[0] = pltpu.async_copy(
            msg_hbm.at[pl.ds(wid * EPW + 0 * CHUNK, CHUNK)], bufs[0], rsems[0])
        for ch in range(NCH):
            b = ch % 2
            nb = (ch + 1) % 2
            if ch + 1 < NCH:
                if ch >= 1:
                    sc[ch - 1].wait()
                r[ch + 1] = pltpu.async_copy(
                    msg_hbm.at[pl.ds(wid * EPW + (ch + 1) * CHUNK, CHUNK)],
                    bufs[nb], rsems[nb])
            r[ch].wait()
            sc[ch] = pltpu.async_copy(
                bufs[b], agg_sh.at[idx_v.at[ch]], ssems[b], add=True)
        sc[NCH - 2].wait()
        sc[NCH - 1].wait()
        plsc.subcore_barrier()

        def out_body(p, carry):
            pltpu.sync_copy(agg_sh.at[pl.ds((s * NSP + p) * SST, SST)], stage_v)
            pltpu.sync_copy(stage_v, agg_hbm.at[c, pl.ds((s * NSP + p) * SST, SST)])
            return carry

        lax.fori_loop(0, NSP, out_body, 0)

    return body_fn


def _sc_scatter(msg, idx2d, zeros_sl):
    return _sc_scatter_kernel()(msg, idx2d, zeros_sl)


def _eye(n):
    r = lax.broadcasted_iota(jnp.int32, (n, n), 0)
    c = lax.broadcasted_iota(jnp.int32, (n, n), 1)
    return (r == c).astype(jnp.float32)


def _rep4(o):
    return jnp.concatenate([o, o, o, o], axis=1)


def _tc_prep(x_p, W, b_row):
    def body(x_ref, w_ref, b_ref, o_ref):
        o = jax.nn.relu(
            jnp.dot(x_ref[...], w_ref[...], preferred_element_type=jnp.float32)
            + b_ref[...]
        )
        o_ref[...] = _rep4(o)

    return pl.pallas_call(
        body,
        out_shape=jax.ShapeDtypeStruct((NPAD, LW), jnp.float32),
    )(x_p, W, b_row)


def _tc_edget(ea_p, W1, b1_row):
    blk = 2048

    def body(ea_ref, w_ref, b_ref, o_ref):
        t = jax.nn.relu(
            jnp.dot(ea_ref[...], w_ref[...], preferred_element_type=jnp.float32)
            + b_ref[...]
        )
        tT = lax.dot_general(_eye(H), t, (((1,), (1,)), ((), ())),
                             preferred_element_type=jnp.float32)
        ones = jnp.ones((1, blk), jnp.float32)
        zeros = jnp.zeros((7, blk), jnp.float32)
        o_ref[...] = jnp.concatenate([tT, ones, zeros], axis=0)

    return pl.pallas_call(
        body,
        out_shape=jax.ShapeDtypeStruct((40, EP), jnp.float32),
        grid=(EP // blk,),
        in_specs=[
            pl.BlockSpec((blk, EDGE_DIM), lambda j: (j, 0)),
            pl.BlockSpec((EDGE_DIM, H), lambda j: (0, 0)),
            pl.BlockSpec((1, H), lambda j: (0, 0)),
        ],
        out_specs=pl.BlockSpec((40, blk), lambda j: (0, j)),
    )(ea_p, W1, b1_row)


def _tc_msg(tT, u, W2b):
    def body(t_ref, u_ref, w_ref, o_ref):
        ew = jnp.dot(w_ref[...], t_ref[...], preferred_element_type=jnp.float32)
        ub = u_ref[:, 0:H]
        uT = lax.dot_general(_eye(H), ub, (((1,), (1,)), ((), ())),
                             preferred_element_type=jnp.float32)
        acc = uT[0:1, :] * ew[0:H, :]
        for i in range(1, H):
            acc = acc + uT[i:i + 1, :] * ew[i * H:(i + 1) * H, :]
        msg = lax.dot_general(acc, _eye(H), (((0,), (0,)), ((), ())),
                              preferred_element_type=jnp.float32)
        o_ref[...] = jnp.concatenate(
            [msg, jnp.zeros((EB, LW - H), jnp.float32)], axis=1)

    return pl.pallas_call(
        body,
        out_shape=jax.ShapeDtypeStruct((EP, LW), jnp.float32),
        grid=(EP // EB,),
        in_specs=[
            pl.BlockSpec((40, EB), lambda j: (0, j)),
            pl.BlockSpec((EB, LW), lambda j: (j, 0)),
            pl.BlockSpec((H * H, 40), lambda j: (0, 0)),
        ],
        out_specs=pl.BlockSpec((EB, LW), lambda j: (j, 0)),
    )(tT, u, W2b)


def _tc_update(out4, h4, aggP, degP, conv_root, cb_row, WihT, WhhT, bih_row, bhh_row):
    def body(o_ref, h_ref, a_ref, d_ref, cr_ref, cb_ref, wi_ref, wh_ref, bi_ref, bh_ref, out_ref):
        agg = a_ref[0, :, 0:H] + a_ref[1, :, 0:H]
        deg = d_ref[0, :, 0:1] + d_ref[1, :, 0:1]
        rdeg = 1.0 / jnp.maximum(deg, 1.0)
        o = o_ref[:, 0:H]
        hh = h_ref[:, 0:H]
        m = jax.nn.relu(
            jnp.dot(o, cr_ref[...], preferred_element_type=jnp.float32)
            + agg * rdeg + cb_ref[...]
        )
        gi = jnp.dot(m, wi_ref[...], preferred_element_type=jnp.float32) + bi_ref[...]
        gh = jnp.dot(hh, wh_ref[...], preferred_element_type=jnp.float32) + bh_ref[...]
        r = jax.nn.sigmoid(gi[:, 0:H] + gh[:, 0:H])
        z = jax.nn.sigmoid(gi[:, H:2 * H] + gh[:, H:2 * H])
        n = jnp.tanh(gi[:, 2 * H:3 * H] + r * gh[:, 2 * H:3 * H])
        out_ref[...] = _rep4((1.0 - z) * n + z * hh)

    return pl.pallas_call(
        body,
        out_shape=jax.ShapeDtypeStruct((NPAD, LW), jnp.float32),
    )(out4, h4, aggP, degP, conv_root, cb_row, WihT, WhhT, bih_row, bhh_row)


def _tc_final(out4, batch_col, batch_row,
              s2s_WihT, s2s_WhhT, s2s_bih_row, s2s_bhh_row,
              mem_WihT, mem_WhhT, mem_bih_row, mem_bhh_row,
              mlp_W1, mlp_b1_row, mlp_W2, mlp_b2_row):
    def body(o_ref, bc_ref, br_ref,
             swi_ref, swh_ref, sbi_ref, sbh_ref,
             mwi_ref, mwh_ref, mbi_ref, mbh_ref,
             w1_ref, b1_ref, w2_ref, b2_ref,
             v_ref, hx_ref, cx_ref):
        o = o_ref[:, 0:H]
        bc = bc_ref[...]
        gcol = lax.broadcasted_iota(jnp.int32, (NPAD, GP), 1)
        OH = (bc == gcol).astype(jnp.float32)
        br = br_ref[...]
        grow = lax.broadcasted_iota(jnp.int32, (GP, NPAD), 0)
        OHT = (br == grow).astype(jnp.float32)

        qh = jnp.zeros((GP, H), jnp.float32)
        qc = jnp.zeros((GP, H), jnp.float32)
        q_star = jnp.zeros((GP, 2 * H), jnp.float32)
        for _ in range(6):
            g = (jnp.dot(q_star, swi_ref[...], preferred_element_type=jnp.float32)
                 + sbi_ref[...]
                 + jnp.dot(qh, swh_ref[...], preferred_element_type=jnp.float32)
                 + sbh_ref[...])
            ig = jax.nn.sigmoid(g[:, 0:H])
            fg = jax.nn.sigmoid(g[:, H:2 * H])
            gg = jnp.tanh(g[:, 2 * H:3 * H])
            og = jax.nn.sigmoid(g[:, 3 * H:4 * H])
            qc = fg * qc + ig * gg
            qh = og * jnp.tanh(qc)

            qhb = jnp.dot(OH, qh, preferred_element_type=jnp.float32)
            e_col = jnp.sum(o * qhb, axis=1, keepdims=True)
            masked = jnp.where(OH > 0.0, e_col, -1e30)
            m_row = jnp.max(masked, axis=0, keepdims=True)
            m_row = jnp.where(m_row < -1e29, 0.0, m_row)
            mb = jnp.sum(OH * m_row, axis=1, keepdims=True)
            ex = jnp.exp(e_col - mb)
            den_row = jnp.sum(OH * ex, axis=0, keepdims=True)
            denb = jnp.sum(OH * den_row, axis=1, keepdims=True)
            a_col = ex / (denb + 1e-16)
            r = jnp.dot(OHT, a_col * o, preferred_element_type=jnp.float32)
            q_star = jnp.concatenate([qh, r], axis=1)

        g = (jnp.dot(q_star, mwi_ref[...], preferred_element_type=jnp.float32)
             + mbi_ref[...] + mbh_ref[...])
        ig = jax.nn.sigmoid(g[:, 0:H])
        fg = jax.nn.sigmoid(g[:, H:2 * H])
        gg = jnp.tanh(g[:, 2 * H:3 * H])
        og = jax.nn.sigmoid(g[:, 3 * H:4 * H])
        cx = ig * gg
        hx = og * jnp.tanh(cx)
        v = (jnp.dot(jax.nn.relu(
                jnp.dot(hx, w1_ref[...], preferred_element_type=jnp.float32)
                + b1_ref[...]),
             w2_ref[...], preferred_element_type=jnp.float32) + b2_ref[...])
        v_ref[...] = v[0:N_GRAPHS, :]
        hx_ref[...] = hx[0:N_GRAPHS, :]
        cx_ref[...] = cx[0:N_GRAPHS, :]

    return pl.pallas_call(
        body,
        out_shape=(
            jax.ShapeDtypeStruct((N_GRAPHS, 1), jnp.float32),
            jax.ShapeDtypeStruct((N_GRAPHS, H), jnp.float32),
            jax.ShapeDtypeStruct((N_GRAPHS, H), jnp.float32),
        ),
    )(out4, batch_col, batch_row,
      s2s_WihT, s2s_WhhT, s2s_bih_row, s2s_bhh_row,
      mem_WihT, mem_WhhT, mem_bih_row, mem_bhh_row,
      mlp_W1, mlp_b1_row, mlp_W2, mlp_b2_row)


def kernel(x, edge_index, edge_attr, batch, lin0_W, lin0_b, enn_W1, enn_b1, enn_W2, enn_b2, conv_root, conv_bias, gru_Wih, gru_Whh, gru_bih, gru_bhh, s2s_Wih, s2s_Whh, s2s_bih, s2s_bhh, mem_Wih, mem_Whh, mem_bih, mem_bhh, mlp_W1, mlp_b1, mlp_W2, mlp_b2):
    f32 = jnp.float32
    src = edge_index[0]
    dst = edge_index[1]
    pad_e = EP - N_EDGES
    src_p = jnp.concatenate([src, jnp.zeros((pad_e,), jnp.int32)]).reshape(NW * NCH, CHUNK)
    trash = N_NODES + (jnp.arange(pad_e, dtype=jnp.int32) % (NPAD - N_NODES))
    dst_p = jnp.concatenate([dst, trash]).reshape(NW * NCH, CHUNK)
    x_p = jnp.concatenate([x, jnp.zeros((NPAD - N_NODES, NODE_DIM), f32)])
    ea_p = jnp.concatenate([edge_attr, jnp.zeros((pad_e, EDGE_DIM), f32)])
    pad_n = NPAD - N_NODES
    batch_p = jnp.concatenate(
        [batch, N_GRAPHS + (jnp.arange(pad_n, dtype=jnp.int32) % (GP - N_GRAPHS))])
    batch_col = batch_p[:, None]
    batch_row = batch_p[None, :]

    W2b = jnp.concatenate(
        [enn_W2.T, enn_b2[:, None], jnp.zeros((H * H, 7), f32)], axis=1)
    ones_ep = jnp.ones((EP, LW), f32)
    zeros_sl = jnp.zeros((SST, LW), f32)

    out = _tc_prep(x_p, lin0_W, lin0_b.reshape(1, H))
    tT = _tc_edget(ea_p, enn_W1, enn_b1.reshape(1, H))
    degP = _sc_scatter(ones_ep, dst_p, zeros_sl)

    h = out
    gWihT = gru_Wih.T
    gWhhT = gru_Whh.T
    for _ in range(6):
        u = _sc_gather(out, src_p)
        msg = _tc_msg(tT, u, W2b)
        aggP = _sc_scatter(msg, dst_p, zeros_sl)
        out = _tc_update(out, h, aggP, degP, conv_root, conv_bias.reshape(1, H),
                         gWihT, gWhhT, gru_bih.reshape(1, 3 * H), gru_bhh.reshape(1, 3 * H))
        h = out

    v, hx, cx = _tc_final(
        out, batch_col, batch_row,
        s2s_Wih.T, s2s_Whh.T, s2s_bih.reshape(1, 4 * H), s2s_bhh.reshape(1, 4 * H),
        mem_Wih.T, mem_Whh.T, mem_bih.reshape(1, 4 * H), mem_bhh.reshape(1, 4 * H),
        mlp_W1, mlp_b1.reshape(1, H), mlp_W2, mlp_b2.reshape(1, 1))
    return v[None], hx[None], cx[None]
